# baseline (speedup 1.0000x reference)
import jax
import jax.numpy as jnp
from jax import lax
from jax.experimental import pallas as pl
from jax.experimental.pallas import tpu as pltpu
from jax.experimental.pallas import tpu_sc as plsc

N_ROWS = 4096
N_FIELDS = 26
N_CLASSES = 1000
N_BANDS = N_FIELDS * N_CLASSES // 8
BANDS_PER_FIELD = N_CLASSES // 8
BAND_W = 32 * 8 * 128
NC = 2
NS = 16
NW = NC * NS


def _scan_scatter(x2_v, buf, off_x, band, value):
    k0 = (band % BANDS_PER_FIELD) * 8

    def body(i, carry):
        rv = x2_v[pl.ds(off_x + i * 16, 16)]
        idx = jnp.where(rv == 0, 0, rv - 1)
        rel = idx - k0
        m = (rel >= 0) & (rel < 8)
        off = ((i // 8) * 1024 + rel * 128 + (i % 8) * 16
               + lax.iota(jnp.int32, 16))
        off = jnp.where(m, off, 0)
        plsc.store_scatter(buf, [off], jnp.full((16,), value, jnp.float32),
                           mask=m)
        return carry
    lax.fori_loop(0, N_ROWS // 16, body, 0)


def _sc_body(xt_hbm, y_hbm, x2_v, buf0, buf1, sem0, sem1):
    c = lax.axis_index("c")
    s = lax.axis_index("s")
    wid = s * NC + c
    start = wid * N_BANDS // NW
    end = (wid + 1) * N_BANDS // NW
    j_lo = start // BANDS_PER_FIELD
    j_hi = (end - 1) // BANDS_PER_FIELD
    j_split = (j_lo + 1) * BANDS_PER_FIELD

    pltpu.sync_copy(xt_hbm.at[pl.ds(j_lo * N_ROWS, N_ROWS)],
                    x2_v.at[pl.ds(0, N_ROWS)])
    pltpu.sync_copy(xt_hbm.at[pl.ds(j_hi * N_ROWS, N_ROWS)],
                    x2_v.at[pl.ds(N_ROWS, N_ROWS)])

    bufs = (buf0, buf1)
    sems = (sem0, sem1)

    def _zero(i, carry):
        buf0[pl.ds(i * 16, 16)] = jnp.zeros((16,), jnp.float32)
        buf1[pl.ds(i * 16, 16)] = jnp.zeros((16,), jnp.float32)
        return carry
    lax.fori_loop(0, BAND_W // 16, _zero, 0)

    def _pair(it, carry):
        for sub in range(2):
            band = start + it * 2 + sub
            in_range = band < end

            @pl.when(in_range & (band - start >= 2))
            def _():
                pltpu.make_async_copy(
                    y_hbm.at[pl.ds(0, BAND_W)], bufs[sub], sems[sub]).wait()
                b_prev = band - 2
                off_prev = jnp.where(b_prev >= j_split, N_ROWS, 0)
                _scan_scatter(x2_v, bufs[sub], off_prev, b_prev, 0.0)

            @pl.when(in_range)
            def _():
                off_x = jnp.where(band >= j_split, N_ROWS, 0)
                _scan_scatter(x2_v, bufs[sub], off_x, band, -1.0)
                pltpu.make_async_copy(
                    bufs[sub], y_hbm.at[pl.ds(band * BAND_W, BAND_W)],
                    sems[sub]).start()
        return carry

    n = end - start
    lax.fori_loop(0, (n + 1) // 2, _pair, 0)

    for sub in range(2):
        pltpu.make_async_copy(
            y_hbm.at[pl.ds(0, BAND_W)], bufs[sub], sems[sub]).wait()


def kernel(x):
    mesh = plsc.VectorSubcoreMesh(core_axis_name="c", subcore_axis_name="s")
    kfn = pl.kernel(
        _sc_body,
        mesh=mesh,
        out_type=jax.ShapeDtypeStruct((N_BANDS * BAND_W,), jnp.float32),
        scratch_types=[
            pltpu.VMEM((2 * N_ROWS,), jnp.int32),
            pltpu.VMEM((BAND_W,), jnp.float32),
            pltpu.VMEM((BAND_W,), jnp.float32),
            pltpu.SemaphoreType.DMA,
            pltpu.SemaphoreType.DMA,
        ],
        compiler_params=pltpu.CompilerParams(
            needs_layout_passes=False, use_tc_tiling_on_sc=False),
    )
    xt = x.T.reshape(-1)
    y = kfn(xt)
    y4 = y.reshape(N_BANDS, 32, 8, 128)
    return y4.transpose(1, 3, 0, 2).reshape(N_ROWS, N_FIELDS * N_CLASSES)


# speedup vs baseline: 4.2535x; 4.2535x over previous
import jax
import jax.numpy as jnp
from jax import lax
from jax.experimental import pallas as pl

N_ROWS = 4096
N_FIELDS = 26
N_CLASSES = 1000


def _onehot_t_kernel(xt_ref, y_ref):
    xj = xt_ref[...].reshape(1, N_ROWS)
    idx = jnp.where(xj == 0, 0, xj - 1)
    k = lax.broadcasted_iota(jnp.int32, (N_CLASSES, N_ROWS), 0)
    y_ref[...] = jnp.where(k == idx, -1.0, 0.0)


def kernel(x):
    xt = x.T.reshape(N_FIELDS, 1, N_ROWS)
    y = pl.pallas_call(
        _onehot_t_kernel,
        grid=(N_FIELDS,),
        in_specs=[pl.BlockSpec((1, 1, N_ROWS), lambda j: (j, 0, 0))],
        out_specs=pl.BlockSpec((N_CLASSES, N_ROWS), lambda j: (j, 0)),
        out_shape=jax.ShapeDtypeStruct((N_FIELDS * N_CLASSES, N_ROWS),
                                       jnp.float32),
    )(xt)
    return y.T
